# baseline (speedup 1.0000x reference)
import functools

import jax
import jax.numpy as jnp
from jax import lax
from jax.experimental import pallas as pl
from jax.experimental.pallas import tpu as pltpu
from jax.experimental.pallas import tpu_sc as plsc

N = 10000
E = 320000
D = 128
DE = 16
H = 8
DH = 16
HDH = H * DH
TW = HDH + 16

NC = 2
NS = 16
L = 16

EPC = E // NS
CHUNK = 80
NCHUNK = EPC // CHUNK
NPAD = 10240
RPT = NPAD // NS

BN = 2000
NPREP = 25
BNP = N // NPREP
BEDGE = E // NPREP


def _prep(x_ref, wnf_ref, wdf_ref, wnb_ref, wdb_ref, ef_ref, aef_ref, aeb_ref,
          hf_ref, df_ref, hb_ref, db_ref, sf_ref, sb_ref):
    xb = x_ref[...]
    hf_ref[...] = jnp.dot(xb, wnf_ref[...], preferred_element_type=jnp.float32)
    df_ref[...] = jnp.dot(xb, wdf_ref[...], preferred_element_type=jnp.float32)
    hb_ref[...] = jnp.dot(xb, wnb_ref[...], preferred_element_type=jnp.float32)
    db_ref[...] = jnp.dot(xb, wdb_ref[...], preferred_element_type=jnp.float32)
    efb = ef_ref[...]
    sf_ref[...] = jnp.dot(efb, aef_ref[...], preferred_element_type=jnp.float32)
    sb_ref[...] = jnp.dot(efb, aeb_ref[...], preferred_element_type=jnp.float32)


def _final(af_ref, ab_ref, x_ref, wo_ref, bc_ref, g_ref, o_ref):
    comb = af_ref[:, :HDH] + ab_ref[:, :HDH]
    y = jnp.dot(comb, wo_ref[...], preferred_element_type=jnp.float32)
    y = y + bc_ref[...] + x_ref[...]
    rms = jnp.sqrt(jnp.mean(y * y, axis=-1, keepdims=True) + 1e-6)
    o_ref[...] = y / rms * g_ref[...]


def _sc_edge_kernel(htab_f, dtab_f, setab_f, src_f, dst_f,
                    htab_b, dtab_b, setab_b, src_b, dst_b,
                    acc_f, acc_b,
                    acctab, idxs4, idxd4,
                    hs0, hs1, sd0, sd1, se0, se1,
                    gsem0, gsem1, isem0, isem1):
    c = lax.axis_index("c")
    s = lax.axis_index("s")

    @plsc.parallel_loop(0, CHUNK, unroll=2)
    def _zero_body(r):
        for cidx in range(TW // L):
            hs0[r, pl.ds(cidx * L, L)] = jnp.zeros((L,), jnp.float32)

    for k in range(RPT // CHUNK):
        pltpu.sync_copy(hs0, acctab.at[pl.ds(s * RPT + k * CHUNK, CHUNK)])
    plsc.subcore_barrier()

    def run_direction(htab, dtab, setab, src, dst, out_hbm):
        ebase0 = s * EPC

        def ebase(j):
            return pl.multiple_of(ebase0 + j * CHUNK, 8)

        def load_idx_sync(j, q):
            pltpu.sync_copy(src.at[pl.ds(ebase(j), CHUNK)], idxs4.at[q])
            pltpu.sync_copy(dst.at[pl.ds(ebase(j), CHUNK)], idxd4.at[q])

        def load_idx_async(j, q, isem):
            pltpu.async_copy(src.at[pl.ds(ebase(j), CHUNK)], idxs4.at[q],
                             isem)
            pltpu.async_copy(dst.at[pl.ds(ebase(j), CHUNK)], idxd4.at[q],
                             isem)

        def wait_idx(j, q, isem):
            pltpu.make_async_copy(src.at[pl.ds(ebase(j), CHUNK)],
                                  idxs4.at[q], isem).wait()
            pltpu.make_async_copy(dst.at[pl.ds(ebase(j), CHUNK)],
                                  idxd4.at[q], isem).wait()

        def gathers(j, q, hs_v, sd_v, se_v, gsem, issue):
            cps = [
                (htab.at[idxs4.at[q]], hs_v),
                (dtab.at[idxd4.at[q]], sd_v),
                (setab.at[pl.ds(ebase(j), CHUNK)], se_v),
            ]
            for src_ref, dst_ref in cps:
                if issue:
                    pltpu.async_copy(src_ref, dst_ref, gsem)
                else:
                    pltpu.make_async_copy(src_ref, dst_ref, gsem).wait()

        load_idx_sync(0, 0)
        load_idx_sync(1, 1)
        gathers(0, 0, hs0, sd0, se0, gsem0, True)
        gathers(1, 1, hs1, sd1, se1, gsem1, True)
        load_idx_async(2, 2, isem0)
        load_idx_async(3, 3, isem1)

        def one_iter(j, hs_v, sd_v, se_v, gsem, isem):
            q = lax.rem(j, 4)
            gathers(j, q, hs_v, sd_v, se_v, gsem, False)

            @plsc.parallel_loop(0, CHUNK, unroll=4)
            def edge_body(e):
                sc = (hs_v[e, pl.ds(HDH, L)] + sd_v[e, pl.ds(0, L)]
                      + se_v[e, pl.ds(0, L)])
                sc = jnp.maximum(sc, sc * 0.2)
                w = jnp.exp(sc)
                hs_v[e, pl.ds(HDH, L)] = w
                for h in range(H):
                    hs_v[e, pl.ds(h * DH, DH)] = (
                        hs_v[e, pl.ds(h * DH, DH)] * w[h])
            pltpu.sync_copy(hs_v, acctab.at[idxd4.at[q]], add=True)

            @pl.when(j <= NCHUNK - 3)
            def _():
                q2 = lax.rem(j + 2, 4)
                wait_idx(j + 2, q2, isem)
                gathers(j + 2, q2, hs_v, sd_v, se_v, gsem, True)

            @pl.when(j <= NCHUNK - 5)
            def _():
                load_idx_async(j + 4, q, isem)

        def pair_body(t, carry):
            one_iter(2 * t, hs0, sd0, se0, gsem0, isem0)
            one_iter(2 * t + 1, hs1, sd1, se1, gsem1, isem1)
            return carry

        lax.fori_loop(0, NCHUNK // 2, pair_body, 0)
        plsc.subcore_barrier()

        def node_chunk(k, kcarry):
            rb = pl.multiple_of(s * RPT + k * CHUNK, 8)
            pltpu.sync_copy(acctab.at[pl.ds(rb, CHUNK)], hs0)

            @plsc.parallel_loop(0, CHUNK, unroll=4)
            def node_body(r):
                den = hs0[r, pl.ds(HDH, L)]
                rec = 1.0 / (den + 1e-9)
                for h in range(H):
                    hs0[r, pl.ds(h * DH, DH)] = (
                        hs0[r, pl.ds(h * DH, DH)] * rec[h])
            pltpu.sync_copy(hs0, out_hbm.at[pl.ds(rb, CHUNK)])
            return kcarry

        lax.fori_loop(0, RPT // CHUNK, node_chunk, 0)

    @pl.when(c == 0)
    def _():
        run_direction(htab_f, dtab_f, setab_f, src_f, dst_f, acc_f)

    @pl.when(c == 1)
    def _():
        run_direction(htab_b, dtab_b, setab_b, src_b, dst_b, acc_b)


_sc_call = functools.partial(
    pl.kernel,
    out_type=[jax.ShapeDtypeStruct((NPAD, TW), jnp.float32),
              jax.ShapeDtypeStruct((NPAD, TW), jnp.float32)],
    mesh=plsc.VectorSubcoreMesh(core_axis_name="c", subcore_axis_name="s"),
    compiler_params=pltpu.CompilerParams(use_tc_tiling_on_sc=False),
    scratch_types=[
        pltpu.VMEM_SHARED((NPAD, TW), jnp.float32),
        pltpu.VMEM((4, CHUNK), jnp.int32),
        pltpu.VMEM((4, CHUNK), jnp.int32),
        pltpu.VMEM((CHUNK, TW), jnp.float32),
        pltpu.VMEM((CHUNK, TW), jnp.float32),
        pltpu.VMEM((CHUNK, L), jnp.float32),
        pltpu.VMEM((CHUNK, L), jnp.float32),
        pltpu.VMEM((CHUNK, L), jnp.float32),
        pltpu.VMEM((CHUNK, L), jnp.float32),
        pltpu.SemaphoreType.DMA,
        pltpu.SemaphoreType.DMA,
        pltpu.SemaphoreType.DMA,
        pltpu.SemaphoreType.DMA,
    ],
)


def kernel(node_features, edge_features, edge_indices, edge_indices_reverse,
           Wf, Wef, a_src_f, a_dst_f, a_edge_f, bf,
           Wb, Web, a_src_b, a_dst_b, a_edge_b, bb,
           Wo, bo, gamma):
    f32 = jnp.float32
    x = node_features

    def node_weights(W, a_src, a_dst):
        W2 = W.reshape(D, HDH)
        A_src = jnp.sum(W * a_src[None], axis=-1)
        A_dst = jnp.sum(W * a_dst[None], axis=-1)
        zn = jnp.zeros((D, TW - HDH - H), f32)
        wn = jnp.concatenate([W2, A_src, zn], axis=1)
        wd = jnp.concatenate([A_dst, jnp.zeros((D, L - H), f32)], axis=1)
        return wn, wd

    wn_f, wd_f = node_weights(Wf, a_src_f, a_dst_f)
    wn_b, wd_b = node_weights(Wb, a_src_b, a_dst_b)

    def edge_weights(We, a_edge):
        Ae = jnp.sum(We * a_edge[None], axis=-1)
        return jnp.concatenate([Ae, jnp.zeros((DE, L - H), f32)], axis=1)

    ae_f = edge_weights(Wef, a_edge_f)
    ae_b = edge_weights(Web, a_edge_b)

    htab_f, dtab_f, htab_b, dtab_b, setab_f, setab_b = pl.pallas_call(
        _prep,
        grid=(NPREP,),
        in_specs=[
            pl.BlockSpec((BNP, D), lambda i: (i, 0)),
            pl.BlockSpec((D, TW), lambda i: (0, 0)),
            pl.BlockSpec((D, L), lambda i: (0, 0)),
            pl.BlockSpec((D, TW), lambda i: (0, 0)),
            pl.BlockSpec((D, L), lambda i: (0, 0)),
            pl.BlockSpec((BEDGE, DE), lambda i: (i, 0)),
            pl.BlockSpec((DE, L), lambda i: (0, 0)),
            pl.BlockSpec((DE, L), lambda i: (0, 0)),
        ],
        out_specs=[
            pl.BlockSpec((BNP, TW), lambda i: (i, 0)),
            pl.BlockSpec((BNP, L), lambda i: (i, 0)),
            pl.BlockSpec((BNP, TW), lambda i: (i, 0)),
            pl.BlockSpec((BNP, L), lambda i: (i, 0)),
            pl.BlockSpec((BEDGE, L), lambda i: (i, 0)),
            pl.BlockSpec((BEDGE, L), lambda i: (i, 0)),
        ],
        out_shape=[
            jax.ShapeDtypeStruct((N, TW), f32),
            jax.ShapeDtypeStruct((N, L), f32),
            jax.ShapeDtypeStruct((N, TW), f32),
            jax.ShapeDtypeStruct((N, L), f32),
            jax.ShapeDtypeStruct((E, L), f32),
            jax.ShapeDtypeStruct((E, L), f32),
        ],
    )(x, wn_f, wd_f, wn_b, wd_b, edge_features, ae_f, ae_b)

    acc_f, acc_b = _sc_call(_sc_edge_kernel)(
        htab_f, dtab_f, setab_f,
        edge_indices[0], edge_indices[1],
        htab_b, dtab_b, setab_b,
        edge_indices_reverse[0], edge_indices_reverse[1])

    bconst = ((bf + bb) @ Wo + bo).reshape(1, D)
    gamma2 = gamma.reshape(1, D)

    out = pl.pallas_call(
        _final,
        grid=(N // BN,),
        in_specs=[
            pl.BlockSpec((BN, TW), lambda i: (i, 0)),
            pl.BlockSpec((BN, TW), lambda i: (i, 0)),
            pl.BlockSpec((BN, D), lambda i: (i, 0)),
            pl.BlockSpec((D, D), lambda i: (0, 0)),
            pl.BlockSpec((1, D), lambda i: (0, 0)),
            pl.BlockSpec((1, D), lambda i: (0, 0)),
        ],
        out_specs=pl.BlockSpec((BN, D), lambda i: (i, 0)),
        out_shape=jax.ShapeDtypeStruct((N, D), f32),
    )(acc_f, acc_b, x, Wo, bconst, gamma2)

    return out


# speedup vs baseline: 1.6693x; 1.6693x over previous
import functools

import jax
import jax.numpy as jnp
from jax import lax
from jax.experimental import pallas as pl
from jax.experimental.pallas import tpu as pltpu
from jax.experimental.pallas import tpu_sc as plsc

N = 10000
E = 320000
D = 128
DE = 16
H = 8
DH = 16
HDH = H * DH

NC = 2
NS = 16
L = 16

EPC = E // NS
CHUNK = 80
NCHUNK = EPC // CHUNK
NPAD = 10240
RPT = NPAD // NS

BN = 2000
E8 = E // 8
NPREP = 5
BNP = N // NPREP
BEDGE = E8 // NPREP


def _prep(x_ref, whf_ref, wsf_ref, wdf_ref, whb_ref, wsb_ref, wdb_ref,
          ef_ref, aef_ref, aeb_ref,
          hf_ref, sf_ref, df_ref, hb_ref, sb_ref, db_ref,
          ef_out_f, ef_out_b):
    xb = x_ref[...]
    hf_ref[...] = jnp.dot(xb, whf_ref[...], preferred_element_type=jnp.float32)
    sf_ref[...] = jnp.dot(xb, wsf_ref[...], preferred_element_type=jnp.float32)
    df_ref[...] = jnp.dot(xb, wdf_ref[...], preferred_element_type=jnp.float32)
    hb_ref[...] = jnp.dot(xb, whb_ref[...], preferred_element_type=jnp.float32)
    sb_ref[...] = jnp.dot(xb, wsb_ref[...], preferred_element_type=jnp.float32)
    db_ref[...] = jnp.dot(xb, wdb_ref[...], preferred_element_type=jnp.float32)
    efb = ef_ref[...]
    ef_out_f[...] = jnp.dot(efb, aef_ref[...],
                            preferred_element_type=jnp.float32)
    ef_out_b[...] = jnp.dot(efb, aeb_ref[...],
                            preferred_element_type=jnp.float32)


def _final(af_ref, ab_ref, x_ref, wo_ref, bc_ref, g_ref, o_ref):
    comb = af_ref[...] + ab_ref[...]
    y = jnp.dot(comb, wo_ref[...], preferred_element_type=jnp.float32)
    y = y + bc_ref[...] + x_ref[...]
    rms = jnp.sqrt(jnp.mean(y * y, axis=-1, keepdims=True) + 1e-6)
    o_ref[...] = y / rms * g_ref[...]


def _sc_edge_kernel(h_f, stab_f, dtab_f, setab_f, ei_f,
                    h_b, stab_b, dtab_b, setab_b, ei_b,
                    out_f, out_b,
                    acc128, denom, idxs4, idxd4,
                    hs0, hs1, ss0, ss1, sd0, sd1, se0, se1, w0, w1,
                    gsem0, gsem1, isem0, isem1):
    c = lax.axis_index("c")
    s = lax.axis_index("s")

    @plsc.parallel_loop(0, CHUNK, unroll=2)
    def _zero_body(r):
        for cidx in range(HDH // L):
            hs0[r, pl.ds(cidx * L, L)] = jnp.zeros((L,), jnp.float32)
        w0[r, pl.ds(0, L)] = jnp.zeros((L,), jnp.float32)

    for k in range(RPT // CHUNK):
        pltpu.sync_copy(hs0, acc128.at[pl.ds(s * RPT + k * CHUNK, CHUNK)])
        pltpu.sync_copy(w0, denom.at[pl.ds(s * RPT + k * CHUNK, CHUNK)])
    plsc.subcore_barrier()

    def run_direction(htab, stab, dtab, setab, ei, out_hbm):
        ebase0 = s * EPC

        def ebase(j):
            return pl.multiple_of(ebase0 + j * CHUNK, 8)

        def load_idx_sync(j, q):
            pltpu.sync_copy(ei.at[0, pl.ds(ebase(j), CHUNK)], idxs4.at[q])
            pltpu.sync_copy(ei.at[1, pl.ds(ebase(j), CHUNK)], idxd4.at[q])

        def load_idx_async(j, q, isem):
            pltpu.async_copy(ei.at[0, pl.ds(ebase(j), CHUNK)], idxs4.at[q],
                             isem)
            pltpu.async_copy(ei.at[1, pl.ds(ebase(j), CHUNK)], idxd4.at[q],
                             isem)

        def wait_idx(j, q, isem):
            pltpu.make_async_copy(ei.at[0, pl.ds(ebase(j), CHUNK)],
                                  idxs4.at[q], isem).wait()
            pltpu.make_async_copy(ei.at[1, pl.ds(ebase(j), CHUNK)],
                                  idxd4.at[q], isem).wait()

        def gathers(j, q, hs_v, ss_v, sd_v, se_v, gsem, issue):
            cps = [
                (htab.at[idxs4.at[q]], hs_v),
                (stab.at[idxs4.at[q]], ss_v),
                (dtab.at[idxd4.at[q]], sd_v),
                (setab.at[pl.ds(ebase(j), CHUNK)], se_v),
            ]
            for src_ref, dst_ref in cps:
                if issue:
                    pltpu.async_copy(src_ref, dst_ref, gsem)
                else:
                    pltpu.make_async_copy(src_ref, dst_ref, gsem).wait()

        load_idx_sync(0, 0)
        load_idx_sync(1, 1)
        gathers(0, 0, hs0, ss0, sd0, se0, gsem0, True)
        gathers(1, 1, hs1, ss1, sd1, se1, gsem1, True)
        load_idx_async(2, 2, isem0)
        load_idx_async(3, 3, isem1)

        def one_iter(j, hs_v, ss_v, sd_v, se_v, w_v, gsem, isem):
            q = lax.rem(j, 4)
            gathers(j, q, hs_v, ss_v, sd_v, se_v, gsem, False)

            @plsc.parallel_loop(0, CHUNK, unroll=4)
            def edge_body(e):
                sc = (ss_v[e, pl.ds(0, L)] + sd_v[e, pl.ds(0, L)]
                      + se_v[e, pl.ds(0, L)])
                sc = jnp.maximum(sc, sc * 0.2)
                w = jnp.exp(sc)
                w_v[e, pl.ds(0, L)] = w
                for h in range(H):
                    hs_v[e, pl.ds(h * DH, DH)] = (
                        hs_v[e, pl.ds(h * DH, DH)] * w[h])

            pltpu.sync_copy(hs_v, acc128.at[idxd4.at[q]], add=True)
            pltpu.sync_copy(w_v, denom.at[idxd4.at[q]], add=True)

            @pl.when(j <= NCHUNK - 3)
            def _():
                q2 = lax.rem(j + 2, 4)
                wait_idx(j + 2, q2, isem)
                gathers(j + 2, q2, hs_v, ss_v, sd_v, se_v, gsem, True)

            @pl.when(j <= NCHUNK - 5)
            def _():
                load_idx_async(j + 4, q, isem)

        def pair_body(t, carry):
            one_iter(2 * t, hs0, ss0, sd0, se0, w0, gsem0, isem0)
            one_iter(2 * t + 1, hs1, ss1, sd1, se1, w1, gsem1, isem1)
            return carry

        lax.fori_loop(0, NCHUNK // 2, pair_body, 0)
        plsc.subcore_barrier()

        def node_chunk(k, kcarry):
            rb = pl.multiple_of(s * RPT + k * CHUNK, 8)
            pltpu.sync_copy(acc128.at[pl.ds(rb, CHUNK)], hs0)
            pltpu.sync_copy(denom.at[pl.ds(rb, CHUNK)], w0)

            @plsc.parallel_loop(0, CHUNK, unroll=4)
            def node_body(r):
                den = w0[r, pl.ds(0, L)]
                rec = 1.0 / (den + 1e-9)
                for h in range(H):
                    hs0[r, pl.ds(h * DH, DH)] = (
                        hs0[r, pl.ds(h * DH, DH)] * rec[h])

            pltpu.sync_copy(hs0, out_hbm.at[pl.ds(rb, CHUNK)])
            return kcarry

        lax.fori_loop(0, RPT // CHUNK, node_chunk, 0)

    @pl.when(c == 0)
    def _():
        run_direction(h_f, stab_f, dtab_f, setab_f, ei_f, out_f)

    @pl.when(c == 1)
    def _():
        run_direction(h_b, stab_b, dtab_b, setab_b, ei_b, out_b)


_sc_call = functools.partial(
    pl.kernel,
    out_type=[jax.ShapeDtypeStruct((NPAD, HDH), jnp.float32),
              jax.ShapeDtypeStruct((NPAD, HDH), jnp.float32)],
    mesh=plsc.VectorSubcoreMesh(core_axis_name="c", subcore_axis_name="s"),
    compiler_params=pltpu.CompilerParams(use_tc_tiling_on_sc=False),
    scratch_types=[
        pltpu.VMEM_SHARED((NPAD, HDH), jnp.float32),
        pltpu.VMEM_SHARED((NPAD, L), jnp.float32),
        pltpu.VMEM((4, CHUNK), jnp.int32),
        pltpu.VMEM((4, CHUNK), jnp.int32),
        pltpu.VMEM((CHUNK, HDH), jnp.float32),
        pltpu.VMEM((CHUNK, HDH), jnp.float32),
        pltpu.VMEM((CHUNK, L), jnp.float32),
        pltpu.VMEM((CHUNK, L), jnp.float32),
        pltpu.VMEM((CHUNK, L), jnp.float32),
        pltpu.VMEM((CHUNK, L), jnp.float32),
        pltpu.VMEM((CHUNK, L), jnp.float32),
        pltpu.VMEM((CHUNK, L), jnp.float32),
        pltpu.VMEM((CHUNK, L), jnp.float32),
        pltpu.VMEM((CHUNK, L), jnp.float32),
        pltpu.SemaphoreType.DMA,
        pltpu.SemaphoreType.DMA,
        pltpu.SemaphoreType.DMA,
        pltpu.SemaphoreType.DMA,
    ],
)


def kernel(node_features, edge_features, edge_indices, edge_indices_reverse,
           Wf, Wef, a_src_f, a_dst_f, a_edge_f, bf,
           Wb, Web, a_src_b, a_dst_b, a_edge_b, bb,
           Wo, bo, gamma):
    f32 = jnp.float32
    x = node_features

    def node_weights(W, a_src, a_dst):
        W2 = W.reshape(D, HDH)
        A_src = jnp.sum(W * a_src[None], axis=-1)
        A_dst = jnp.sum(W * a_dst[None], axis=-1)
        zpad = jnp.zeros((D, L - H), f32)
        ws = jnp.concatenate([A_src, zpad], axis=1)
        wd = jnp.concatenate([A_dst, zpad], axis=1)
        return W2, ws, wd

    wh_f, ws_f, wd_f = node_weights(Wf, a_src_f, a_dst_f)
    wh_b, ws_b, wd_b = node_weights(Wb, a_src_b, a_dst_b)

    def edge_weights(We, a_edge):
        Ae = jnp.sum(We * a_edge[None], axis=-1)
        ae = jnp.concatenate([Ae, jnp.zeros((DE, L - H), f32)], axis=1)
        return jnp.kron(jnp.eye(8, dtype=f32), ae)

    ae_f = edge_weights(Wef, a_edge_f)
    ae_b = edge_weights(Web, a_edge_b)
    ef2 = edge_features.reshape(E8, 8 * DE)

    (h128_f, stab_f, dtab_f, h128_b, stab_b, dtab_b,
     setab2_f, setab2_b) = pl.pallas_call(
        _prep,
        grid=(NPREP,),
        in_specs=[
            pl.BlockSpec((BNP, D), lambda i: (i, 0)),
            pl.BlockSpec((D, HDH), lambda i: (0, 0)),
            pl.BlockSpec((D, L), lambda i: (0, 0)),
            pl.BlockSpec((D, L), lambda i: (0, 0)),
            pl.BlockSpec((D, HDH), lambda i: (0, 0)),
            pl.BlockSpec((D, L), lambda i: (0, 0)),
            pl.BlockSpec((D, L), lambda i: (0, 0)),
            pl.BlockSpec((BEDGE, 8 * DE), lambda i: (i, 0)),
            pl.BlockSpec((8 * DE, 8 * L), lambda i: (0, 0)),
            pl.BlockSpec((8 * DE, 8 * L), lambda i: (0, 0)),
        ],
        out_specs=[
            pl.BlockSpec((BNP, HDH), lambda i: (i, 0)),
            pl.BlockSpec((BNP, L), lambda i: (i, 0)),
            pl.BlockSpec((BNP, L), lambda i: (i, 0)),
            pl.BlockSpec((BNP, HDH), lambda i: (i, 0)),
            pl.BlockSpec((BNP, L), lambda i: (i, 0)),
            pl.BlockSpec((BNP, L), lambda i: (i, 0)),
            pl.BlockSpec((BEDGE, 8 * L), lambda i: (i, 0)),
            pl.BlockSpec((BEDGE, 8 * L), lambda i: (i, 0)),
        ],
        out_shape=[
            jax.ShapeDtypeStruct((N, HDH), f32),
            jax.ShapeDtypeStruct((N, L), f32),
            jax.ShapeDtypeStruct((N, L), f32),
            jax.ShapeDtypeStruct((N, HDH), f32),
            jax.ShapeDtypeStruct((N, L), f32),
            jax.ShapeDtypeStruct((N, L), f32),
            jax.ShapeDtypeStruct((E8, 8 * L), f32),
            jax.ShapeDtypeStruct((E8, 8 * L), f32),
        ],
    )(x, wh_f, ws_f, wd_f, wh_b, ws_b, wd_b, ef2, ae_f, ae_b)
    setab_f = setab2_f.reshape(E, L)
    setab_b = setab2_b.reshape(E, L)

    out_f, out_b = _sc_call(_sc_edge_kernel)(
        h128_f, stab_f, dtab_f, setab_f, edge_indices,
        h128_b, stab_b, dtab_b, setab_b, edge_indices_reverse)

    bconst = ((bf + bb) @ Wo + bo).reshape(1, D)
    gamma2 = gamma.reshape(1, D)

    out = pl.pallas_call(
        _final,
        grid=(N // BN,),
        in_specs=[
            pl.BlockSpec((BN, HDH), lambda i: (i, 0)),
            pl.BlockSpec((BN, HDH), lambda i: (i, 0)),
            pl.BlockSpec((BN, D), lambda i: (i, 0)),
            pl.BlockSpec((D, D), lambda i: (0, 0)),
            pl.BlockSpec((1, D), lambda i: (0, 0)),
            pl.BlockSpec((1, D), lambda i: (0, 0)),
        ],
        out_specs=pl.BlockSpec((BN, D), lambda i: (i, 0)),
        out_shape=jax.ShapeDtypeStruct((N, D), f32),
    )(out_f, out_b, x, Wo, bconst, gamma2)

    return out
